# Initial kernel scaffold; baseline (speedup 1.0000x reference)
import jax
import jax.numpy as jnp
from jax.experimental import pallas as pl


def kernel(grid, color_embed, pos_embed, gamma, beta):
    raise NotImplementedError("write your pallas kernel here")


# speedup vs baseline: 12.9369x; 12.9369x over previous
import functools

import jax
import jax.numpy as jnp
from jax.experimental import pallas as pl
from jax.experimental.pallas import tpu as pltpu

_BB = 16


def _tab_kernel(ce_ref, pos_ref, gam_ref, bet_ref, tab_ref):
    ce = ce_ref[...]
    pos = pos_ref[...]
    x = ce[:, None, :] + pos[None, :, :]
    mu = jnp.mean(x, axis=-1, keepdims=True)
    xc = x - mu
    var = jnp.mean(xc * xc, axis=-1, keepdims=True)
    xn = xc * jax.lax.rsqrt(var + 1e-5)
    tab_ref[...] = xn * gam_ref[0][None, None, :] + bet_ref[0][None, None, :]


def _select_kernel(g_ref, tab_ref, out_ref, *, num_colors):
    g = g_ref[...].astype(jnp.int32)
    tab = tab_ref[...]
    acc = jnp.broadcast_to(tab[0][None], out_ref.shape)
    for c in range(1, num_colors):
        acc = jnp.where(g == c, tab[c][None], acc)
    out_ref[...] = acc


def kernel(grid, color_embed, pos_embed, gamma, beta):
    B, H, W = grid.shape
    P = H * W
    C, D = color_embed.shape

    g3 = jnp.broadcast_to(grid.reshape(B, P, 1).astype(jnp.int8), (B, P, D))
    posf = pos_embed[0, :H, :W, :].reshape(P, D)

    tab = pl.pallas_call(
        _tab_kernel,
        out_shape=jax.ShapeDtypeStruct((C, P, D), jnp.float32),
    )(color_embed, posf, gamma.reshape(1, D), beta.reshape(1, D))

    out = pl.pallas_call(
        functools.partial(_select_kernel, num_colors=C),
        grid=(B // _BB,),
        in_specs=[
            pl.BlockSpec((_BB, P, D), lambda i: (i, 0, 0)),
            pl.BlockSpec((C, P, D), lambda i: (0, 0, 0)),
        ],
        out_specs=pl.BlockSpec((_BB, P, D), lambda i: (i, 0, 0)),
        out_shape=jax.ShapeDtypeStruct((B, P, D), jnp.float32),
    )(g3, tab)
    return out


# speedup vs baseline: 18.4615x; 1.4270x over previous
import functools

import jax
import jax.numpy as jnp
from jax.experimental import pallas as pl
from jax.experimental.pallas import tpu as pltpu

_BB = 32


def _tab_kernel(ce_ref, pos_ref, gam_ref, bet_ref, tab_ref):
    ce = ce_ref[...]
    pos = pos_ref[...]
    x = ce[:, None, :] + pos[None, :, :]
    mu = jnp.mean(x, axis=-1, keepdims=True)
    xc = x - mu
    var = jnp.mean(xc * xc, axis=-1, keepdims=True)
    xn = xc * jax.lax.rsqrt(var + 1e-5)
    tab_ref[...] = xn * gam_ref[0][None, None, :] + bet_ref[0][None, None, :]


def _select_kernel(g_ref, tab_ref, out_ref, *, num_colors):
    g = g_ref[...].astype(jnp.int32)
    tab = tab_ref[...]
    acc = jnp.broadcast_to(tab[0][None], out_ref.shape)
    for c in range(1, num_colors):
        acc = jnp.where(g == c, tab[c][None], acc)
    out_ref[...] = acc


def kernel(grid, color_embed, pos_embed, gamma, beta):
    B, H, W = grid.shape
    P = H * W
    C, D = color_embed.shape

    posf = pos_embed[0, :H, :W, :].reshape(P, D)

    tab = pl.pallas_call(
        _tab_kernel,
        out_shape=jax.ShapeDtypeStruct((C, P, D), jnp.float32),
    )(color_embed, posf, gamma.reshape(1, D), beta.reshape(1, D))

    P2, D2 = P // 2, 2 * D
    g3 = jnp.broadcast_to(
        grid.reshape(B, P2, 2, 1).astype(jnp.int8), (B, P2, 2, D)
    ).reshape(B, P2, D2)
    tabw = tab.reshape(C, P2, D2)

    out = pl.pallas_call(
        functools.partial(_select_kernel, num_colors=C),
        grid=(B // _BB,),
        in_specs=[
            pl.BlockSpec((_BB, P2, D2), lambda i: (i, 0, 0)),
            pl.BlockSpec((C, P2, D2), lambda i: (0, 0, 0)),
        ],
        out_specs=pl.BlockSpec((_BB, P2, D2), lambda i: (i, 0, 0)),
        out_shape=jax.ShapeDtypeStruct((B, P2, D2), jnp.float32),
    )(g3, tabw)
    return out.reshape(B, P, D)
